# baseline (speedup 1.0000x reference)
import functools

import jax
import jax.numpy as jnp
from jax import lax
from jax.experimental import pallas as pl
from jax.experimental.pallas import tpu as pltpu
from jax.experimental.pallas import tpu_sc as plsc

_NC = 2
_NS = 16
_NW = _NC * _NS

_CHUNK = 1600


def _positional_encoding(seqlen: int, dmodel: int) -> jnp.ndarray:
    pos = jnp.arange(seqlen, dtype=jnp.float32)[:, None]
    ch = jnp.arange(dmodel, dtype=jnp.float32)[None, :]
    angle = pos * jnp.power(10000.0, -2.0 * ch / float(dmodel))
    even_mask = (jnp.arange(dmodel) % 2 == 0)[None, :]
    return jnp.where(even_mask, jnp.sin(angle), jnp.cos(angle))


@functools.partial(jax.jit, static_argnames=("n_rows", "seqlen"))
def _sc_embed(idx_flat, table, pe, pos_idx, *, n_rows, seqlen):
    dmodel = table.shape[1]
    b_per_w = n_rows // _NW
    n_chunks = b_per_w // _CHUNK
    mesh = plsc.VectorSubcoreMesh(core_axis_name="c", subcore_axis_name="s")

    @functools.partial(
        pl.kernel,
        out_type=jax.ShapeDtypeStruct((n_rows, dmodel), jnp.float32),
        mesh=mesh,
        scratch_types=[
            pltpu.VMEM((_CHUNK,), jnp.int32),
            pltpu.VMEM((_CHUNK,), jnp.int32),
            pltpu.VMEM((_CHUNK, dmodel), jnp.float32),
            pltpu.SemaphoreType.DMA,
            pltpu.SemaphoreType.DMA,
        ],
        compiler_params=pltpu.CompilerParams(use_tc_tiling_on_sc=False),
    )
    def body(table_hbm, idx_hbm, pe_hbm, pos_hbm, out_hbm,
             idx_v, pos_v, rows_v, gsem, psem):
        wid = lax.axis_index("s") * _NC + lax.axis_index("c")
        base = wid * b_per_w
        pltpu.sync_copy(pos_hbm, pos_v)

        def step(c):
            off = base + c * _CHUNK
            pltpu.sync_copy(idx_hbm.at[pl.ds(off, _CHUNK)], idx_v)
            pltpu.async_copy(table_hbm.at[idx_v], rows_v, gsem).wait()
            pltpu.async_copy(pe_hbm.at[pos_v], rows_v, psem, add=True).wait()
            pltpu.sync_copy(rows_v, out_hbm.at[pl.ds(off, _CHUNK)])

        pl.loop(0, n_chunks)(step)

    return body(table, idx_flat, pe, pos_idx)


def kernel(seq, table):
    batch, seqlen = seq.shape
    dmodel = table.shape[1]
    n_rows = batch * seqlen
    idx_flat = seq.reshape(n_rows).astype(jnp.int32)
    pe = _positional_encoding(seqlen, dmodel)
    pos_idx = (jnp.arange(_CHUNK, dtype=jnp.int32) % seqlen)
    out = _sc_embed(idx_flat, table, pe, pos_idx, n_rows=n_rows, seqlen=seqlen)
    return out.reshape(batch, seqlen, dmodel)
